# baseline (speedup 1.0000x reference)
import functools

import jax
import jax.numpy as jnp
from jax import lax
from jax.experimental import pallas as pl
from jax.experimental.pallas import tpu as pltpu
from jax.experimental.pallas import tpu_sc as plsc

NUM_CORES = 2
NUM_SUBCORES = 16
NUM_WORKERS = NUM_CORES * NUM_SUBCORES


def _make_gather(batch: int, seq: int, dim: int):
    assert batch % (2 * NUM_WORKERS) == 0
    pages_per_w = batch // NUM_WORKERS
    mesh = plsc.VectorSubcoreMesh(core_axis_name="c", subcore_axis_name="s")

    @functools.partial(
        pl.kernel,
        mesh=mesh,
        compiler_params=pltpu.CompilerParams(use_tc_tiling_on_sc=False),
        out_type=jax.ShapeDtypeStruct((batch, seq, dim), jnp.float32),
        scratch_types=[
            pltpu.VMEM((seq,), jnp.int32),
            pltpu.VMEM((seq,), jnp.int32),
            pltpu.VMEM((seq, dim), jnp.float32),
            pltpu.VMEM((seq, dim), jnp.float32),
            pltpu.SemaphoreType.DMA,
            pltpu.SemaphoreType.DMA,
            pltpu.SemaphoreType.DMA,
            pltpu.SemaphoreType.DMA,
        ],
    )
    def gather_kernel(table_hbm, idx_hbm, out_hbm, i0, i1, r0, r1, g0, g1, s0, s1):
        wid = lax.axis_index("s") * NUM_CORES + lax.axis_index("c")
        base_b = wid * pages_per_w
        idx_v = (i0, i1)
        rows_v = (r0, r1)
        gsem = (g0, g1)
        ssem = (s0, s1)

        def g_start(p, b):
            bb = base_b + p
            pltpu.sync_copy(idx_hbm.at[bb], idx_v[b])
            pltpu.async_copy(table_hbm.at[idx_v[b]], rows_v[b], gsem[b])

        def g_wait(b):
            pltpu.make_async_copy(
                table_hbm.at[pl.ds(0, seq), :], rows_v[b], gsem[b]).wait()

        def s_start(p, b):
            pltpu.async_copy(rows_v[b], out_hbm.at[base_b + p], ssem[b])

        def s_wait(b):
            pltpu.make_async_copy(rows_v[b], out_hbm.at[base_b], ssem[b]).wait()

        def step(p, b):
            g_wait(b)
            s_start(p, b)
            s_wait(1 - b)
            g_start(p + 1, 1 - b)

        g_start(0, 0)
        g_wait(0)
        s_start(0, 0)
        g_start(1, 1)

        @pl.loop(1, pages_per_w - 1, step=2)
        def _(p):
            step(p, 1)
            step(p + 1, 0)

        g_wait(1)
        s_start(pages_per_w - 1, 1)
        s_wait(0)
        s_wait(1)

    return gather_kernel


def kernel(token_indices, token_embedding_table):
    b, t = token_indices.shape
    v, d = token_embedding_table.shape
    idx = token_indices.astype(jnp.int32)
    return _make_gather(b, t, d)(token_embedding_table, idx)
